# baseline (speedup 1.0000x reference)
import functools

import jax
import jax.numpy as jnp
from jax import lax
from jax.experimental import pallas as pl
from jax.experimental.pallas import tpu as pltpu
from jax.experimental.pallas import tpu_sc as plsc

L = 16
NW = 32
CH = 128
NCH = 50
TPW = CH * NCH
OB = CH // 2 * 128


def _emb_call(tok1d, table2, pos1d):
    mesh = plsc.VectorSubcoreMesh(core_axis_name="c", subcore_axis_name="s")
    n_out = tok1d.shape[0] * 64

    @functools.partial(
        pl.kernel,
        mesh=mesh,
        compiler_params=pltpu.CompilerParams(use_tc_tiling_on_sc=False,
                                             needs_layout_passes=False),
        out_type=jax.ShapeDtypeStruct((n_out,), jnp.float32),
        scratch_types=[
            pltpu.VMEM((TPW,), jnp.int32),
            pltpu.VMEM((NCH, CH), jnp.int32),
            pltpu.VMEM((NCH, CH), jnp.int32),
            pltpu.VMEM((TPW * 2,), jnp.float32),
            pltpu.VMEM((CH, 128), jnp.float32),
            pltpu.VMEM((CH, 128), jnp.float32),
            pltpu.VMEM((OB,), jnp.float32),
            pltpu.VMEM((OB,), jnp.float32),
            pltpu.SemaphoreType.DMA,
            pltpu.SemaphoreType.DMA,
            pltpu.SemaphoreType.DMA,
            pltpu.SemaphoreType.DMA,
        ],
    )
    def k(tok_hbm, tab_hbm, pos_hbm, out_hbm, tok_v, pidx_v, sel_v, pos_v,
          buf0, buf1, obuf0, obuf1, gsem0, gsem1, osem0, osem1):
        wid = lax.axis_index("s") * 2 + lax.axis_index("c")
        bufs = (buf0, buf1)
        obufs = (obuf0, obuf1)
        gsems = (gsem0, gsem1)
        osems = (osem0, osem1)
        iota = lax.iota(jnp.int32, L)

        pltpu.sync_copy(tok_hbm.at[pl.ds(wid * TPW, TPW)], tok_v)
        pltpu.sync_copy(pos_hbm, pos_v)

        def pre_row(r, carry):
            for s in range(CH // L):
                v = tok_v[pl.ds(r * CH + s * L, L)]
                pidx_v[r, pl.ds(s * L, L)] = lax.shift_right_logical(v, 1)
                sel_v[r, pl.ds(s * L, L)] = lax.shift_left(
                    lax.bitwise_and(v, 1), 6)
            return carry

        lax.fori_loop(0, NCH, pre_row, 0)

        def out_off(c):
            return (wid * NCH + c) * OB

        pltpu.async_copy(tab_hbm.at[pidx_v.at[0]], buf0, gsem0)
        pltpu.async_copy(tab_hbm.at[pidx_v.at[1]], buf1, gsem1)

        def super_body(g, carry):
            for p in range(2):
                c = 2 * g + p
                buf, obuf = bufs[p], obufs[p]
                pltpu.make_async_copy(tab_hbm.at[pidx_v.at[c]], buf,
                                      gsems[p]).wait()
                @pl.when(g >= 1)
                def _():
                    pltpu.make_async_copy(obuf, out_hbm.at[pl.ds(out_off(c),
                                                                 OB)],
                                          osems[p]).wait()

                cv = jnp.broadcast_to(c, (L,)).astype(jnp.int32)
                obase = wid * NCH * (CH // 2) + c * (CH // 2)

                def asm_row(ro, carry2):
                    for half in range(2):
                        r = 2 * ro + half
                        rowv = jnp.broadcast_to(r, (L,)).astype(jnp.int32)
                        selb = plsc.load_gather(sel_v, [cv, rowv])
                        posrow = lax.rem(obase + ro, 100)
                        pbase = posrow * 128 + half * 64
                        dbase = ro * 128 + half * 64
                        for j in range(4):
                            colv = selb + (j * L + iota)
                            val = plsc.load_gather(buf, [rowv, colv])
                            pos16 = pos_v[pl.ds(pbase + j * L, L)]
                            obuf[pl.ds(dbase + j * L, L)] = val + pos16
                    return carry2

                lax.fori_loop(0, CH // 2, asm_row, 0)

                pltpu.async_copy(obuf, out_hbm.at[pl.ds(out_off(c), OB)],
                                 osems[p])

                @pl.when(g < (NCH // 2) - 1)
                def _():
                    pltpu.async_copy(tab_hbm.at[pidx_v.at[c + 2]], buf,
                                     gsems[p])
            return carry

        lax.fori_loop(0, NCH // 2, super_body, 0)

        pltpu.make_async_copy(obuf0, out_hbm.at[pl.ds(out_off(NCH - 2), OB)],
                              osem0).wait()
        pltpu.make_async_copy(obuf1, out_hbm.at[pl.ds(out_off(NCH - 1), OB)],
                              osem1).wait()

    return k(tok1d, table2, pos1d)


def kernel(tokens, token_table, position_embeddings):
    batch, n_token = tokens.shape
    tok1d = tokens.astype(jnp.int32).reshape(-1)
    table2 = token_table.reshape(-1, 128)
    pos1d = position_embeddings.reshape(-1)
    out = _emb_call(tok1d, table2, pos1d)
    return out.reshape(batch, n_token, 64)


# speedup vs baseline: 1.0828x; 1.0828x over previous
import functools

import jax
import jax.numpy as jnp
from jax import lax
from jax.experimental import pallas as pl
from jax.experimental.pallas import tpu as pltpu
from jax.experimental.pallas import tpu_sc as plsc

L = 16
NW = 32
CH = 128
NCH = 50
TPW = CH * NCH
D = 64
PER = 200 * D


def _emb_call(tok1d, table, pos1d):
    mesh = plsc.VectorSubcoreMesh(core_axis_name="c", subcore_axis_name="s")
    n_tok = tok1d.shape[0]

    @functools.partial(
        pl.kernel,
        mesh=mesh,
        compiler_params=pltpu.CompilerParams(use_tc_tiling_on_sc=False),
        out_type=jax.ShapeDtypeStruct((n_tok, D), jnp.float32),
        scratch_types=[
            pltpu.VMEM((TPW,), jnp.int32),
            pltpu.VMEM((PER,), jnp.float32),
            pltpu.VMEM((CH, D), jnp.float32),
            pltpu.VMEM((CH, D), jnp.float32),
            pltpu.SemaphoreType.DMA,
            pltpu.SemaphoreType.DMA,
            pltpu.SemaphoreType.DMA,
            pltpu.SemaphoreType.DMA,
        ],
    )
    def k(tok_hbm, tab_hbm, pos_hbm, out_hbm, tok_v, pos_v, buf0, buf1,
          gsem0, gsem1, osem0, osem1):
        wid = lax.axis_index("s") * 2 + lax.axis_index("c")
        bufs = (buf0, buf1)
        gsems = (gsem0, gsem1)
        osems = (osem0, osem1)

        pltpu.sync_copy(tok_hbm.at[pl.ds(wid * TPW, TPW)], tok_v)
        pltpu.sync_copy(pos_hbm, pos_v)

        def row0(c):
            return (wid * NCH + c) * CH

        pltpu.async_copy(tab_hbm.at[tok_v.at[pl.ds(0, CH)]], buf0, gsem0)
        pltpu.async_copy(tab_hbm.at[tok_v.at[pl.ds(CH, CH)]], buf1, gsem1)

        def super_body(g, carry):
            for p in range(2):
                c = 2 * g + p
                buf = bufs[p]
                pltpu.make_async_copy(
                    tab_hbm.at[tok_v.at[pl.ds(c * CH, CH)]], buf,
                    gsems[p]).wait()
                @pl.when(g >= 1)
                def _():
                    pltpu.make_async_copy(buf,
                                          out_hbm.at[pl.ds(row0(c), CH)],
                                          osems[p]).wait()

                po0 = lax.rem(c * (CH * D), PER)

                def add_row(r, po):
                    for u in range(D // L):
                        buf[r, pl.ds(u * L, L)] += pos_v[pl.ds(po + u * L, L)]
                    po = po + D
                    return lax.select(po >= PER, po - PER, po)

                lax.fori_loop(0, CH, add_row, po0)

                pltpu.async_copy(buf, out_hbm.at[pl.ds(row0(c), CH)],
                                 osems[p])

                @pl.when(g < (NCH // 2) - 1)
                def _():
                    pltpu.async_copy(
                        tab_hbm.at[tok_v.at[pl.ds((c + 2) * CH, CH)]], buf,
                        gsems[p])
            return carry

        lax.fori_loop(0, NCH // 2, super_body, 0)

        pltpu.make_async_copy(buf0, out_hbm.at[pl.ds(row0(NCH - 2), CH)],
                              osem0).wait()
        pltpu.make_async_copy(buf1, out_hbm.at[pl.ds(row0(NCH - 1), CH)],
                              osem1).wait()

    return k(tok1d, table, pos1d)


def kernel(tokens, token_table, position_embeddings):
    batch, n_token = tokens.shape
    tok1d = tokens.astype(jnp.int32).reshape(-1)
    pos1d = position_embeddings.reshape(-1)
    out = _emb_call(tok1d, token_table, pos1d)
    return out.reshape(batch, n_token, D)
